# Initial kernel scaffold; baseline (speedup 1.0000x reference)
import jax
import jax.numpy as jnp
from jax.experimental import pallas as pl


def kernel(x, weights, top_weights, top_experts, gate_up_weights, down_weights):
    raise NotImplementedError("write your pallas kernel here")


# speedup vs baseline: 2.6690x; 2.6690x over previous
import functools

import jax
import jax.numpy as jnp
from jax import lax
from jax.experimental import pallas as pl
from jax.experimental.pallas import tpu as pltpu
from jax.experimental.pallas import tpu_sc as plsc

S = 2048
D = 768
E = 64
F = 1536
TM = 128
T_MAX = E + S // TM
P = T_MAX * TM

NC, NS = 2, 16
NW = NC * NS
ROWS_A = P // NW
CHUNK = 64
ROWS_C = S // NW


def _routing_metadata(top_experts):
    e_t = top_experts[:, 0].astype(jnp.int32)
    order = jnp.argsort(e_t, stable=True).astype(jnp.int32)
    sorted_e = jnp.take(e_t, order)
    counts = jnp.zeros((E,), jnp.int32).at[e_t].add(1)
    ntiles = (counts + TM - 1) // TM
    tcum = jnp.cumsum(ntiles)
    tcum_excl = tcum - ntiles
    tile_expert = jnp.minimum(
        jnp.searchsorted(tcum, jnp.arange(T_MAX, dtype=jnp.int32), side="right"),
        E - 1,
    ).astype(jnp.int32)
    counts_excl = jnp.cumsum(counts) - counts
    i = jnp.arange(S, dtype=jnp.int32)
    pos_sorted = TM * jnp.take(tcum_excl, sorted_e) + i - jnp.take(counts_excl, sorted_e)
    src_row = jnp.zeros((P,), jnp.int32).at[pos_sorted].set(order)
    pos = jnp.zeros((S,), jnp.int32).at[order].set(pos_sorted)
    return src_row, pos, tile_expert


def _sc_gather_in(x, src_row, tww):
    mesh = plsc.VectorSubcoreMesh(core_axis_name="c", subcore_axis_name="s")

    @functools.partial(
        pl.kernel,
        mesh=mesh,
        out_type=[
            jax.ShapeDtypeStruct((P, D), jnp.float32),
            jax.ShapeDtypeStruct((P, 128), jnp.float32),
        ],
        scratch_types=[
            pltpu.VMEM((ROWS_A,), jnp.int32),
            pltpu.VMEM((CHUNK, D), jnp.float32),
            pltpu.VMEM((ROWS_A, 128), jnp.float32),
            pltpu.SemaphoreType.DMA,
        ],
    )
    def ka(x_hbm, src_hbm, tw_hbm, xs_hbm, w_hbm, idx_v, rows_v, w_v, sem):
        wid = lax.axis_index("s") * NC + lax.axis_index("c")
        base = wid * ROWS_A
        pltpu.sync_copy(src_hbm.at[pl.ds(base, ROWS_A)], idx_v)
        for c in range(ROWS_A // CHUNK):
            idx_c = idx_v.at[pl.ds(c * CHUNK, CHUNK)]
            pltpu.async_copy(x_hbm.at[idx_c], rows_v, sem).wait()
            pltpu.sync_copy(rows_v, xs_hbm.at[pl.ds(base + c * CHUNK, CHUNK)])
            pltpu.async_copy(
                tw_hbm.at[idx_c], w_v.at[pl.ds(c * CHUNK, CHUNK)], sem
            ).wait()
        pltpu.sync_copy(w_v, w_hbm.at[pl.ds(base, ROWS_A)])

    return ka(x, src_row, tww)


def _sc_gather_out(ys, pos):
    mesh = plsc.VectorSubcoreMesh(core_axis_name="c", subcore_axis_name="s")

    @functools.partial(
        pl.kernel,
        mesh=mesh,
        out_type=jax.ShapeDtypeStruct((S, D), jnp.float32),
        scratch_types=[
            pltpu.VMEM((ROWS_C,), jnp.int32),
            pltpu.VMEM((ROWS_C, D), jnp.float32),
            pltpu.SemaphoreType.DMA,
        ],
    )
    def kc(ys_hbm, pos_hbm, out_hbm, idx_v, rows_v, sem):
        wid = lax.axis_index("s") * NC + lax.axis_index("c")
        base = wid * ROWS_C
        pltpu.sync_copy(pos_hbm.at[pl.ds(base, ROWS_C)], idx_v)
        pltpu.async_copy(ys_hbm.at[idx_v], rows_v, sem).wait()
        pltpu.sync_copy(rows_v, out_hbm.at[pl.ds(base, ROWS_C)])

    return kc(ys, pos)


def _tc_body(te_ref, xs_ref, gu_ref, dn_ref, w_ref, ys_ref):
    xt = xs_ref[...]
    gu = gu_ref[0]
    acts = lax.dot_general(
        xt, gu, (((1,), (1,)), ((), ())), preferred_element_type=jnp.float32
    )
    gate = acts[:, :F]
    up = acts[:, F:]
    act = gate * jax.nn.sigmoid(gate) * up
    dn = dn_ref[0]
    y = lax.dot_general(
        act, dn, (((1,), (1,)), ((), ())), preferred_element_type=jnp.float32
    )
    ys_ref[...] = y * w_ref[0, :, 0][:, None]


def _tc_grouped_matmul(xs, gate_up_weights, down_weights, w2, tile_expert,
                       interpret=False):
    grid_spec = pltpu.PrefetchScalarGridSpec(
        num_scalar_prefetch=1,
        grid=(T_MAX,),
        in_specs=[
            pl.BlockSpec((TM, D), lambda j, te: (j, 0)),
            pl.BlockSpec((1, 2 * F, D), lambda j, te: (te[j], 0, 0)),
            pl.BlockSpec((1, D, F), lambda j, te: (te[j], 0, 0)),
            pl.BlockSpec((1, TM, 128), lambda j, te: (j, 0, 0)),
        ],
        out_specs=pl.BlockSpec((TM, D), lambda j, te: (j, 0)),
    )
    return pl.pallas_call(
        _tc_body,
        grid_spec=grid_spec,
        out_shape=jax.ShapeDtypeStruct((P, D), jnp.float32),
        interpret=interpret,
    )(tile_expert, xs, gate_up_weights, down_weights,
      w2.reshape(T_MAX, TM, 128))


def kernel(x, weights, top_weights, top_experts, gate_up_weights, down_weights):
    del weights
    q_len = x.shape[1]
    xf = x.reshape(S, D)
    tww = jnp.broadcast_to(
        top_weights[:, 0].astype(jnp.float32)[:, None], (S, 128)
    )

    src_row, pos, tile_expert = _routing_metadata(top_experts)
    xs, w = _sc_gather_in(xf, src_row, tww)
    ys = _tc_grouped_matmul(xs, gate_up_weights, down_weights, w, tile_expert)
    out = _sc_gather_out(ys, pos)
    return out.reshape(-1, q_len, D)


# speedup vs baseline: 2.8516x; 1.0684x over previous
import functools

import jax
import jax.numpy as jnp
from jax import lax
from jax.experimental import pallas as pl
from jax.experimental.pallas import tpu as pltpu
from jax.experimental.pallas import tpu_sc as plsc

S = 2048
D = 768
E = 64
F = 1536
TM = 128
T_MAX = E + S // TM
P = T_MAX * TM

NC, NS = 2, 16
NW = NC * NS
ROWS_A = P // NW
CHUNK = 64
ROWS_C = S // NW


def _routing_metadata(top_experts):
    e_t = top_experts[:, 0].astype(jnp.int32)
    order = jnp.argsort(e_t, stable=True).astype(jnp.int32)
    sorted_e = jnp.take(e_t, order)
    counts = jnp.zeros((E,), jnp.int32).at[e_t].add(1)
    ntiles = (counts + TM - 1) // TM
    tcum = jnp.cumsum(ntiles)
    tcum_excl = tcum - ntiles
    tile_expert = jnp.minimum(
        jnp.searchsorted(tcum, jnp.arange(T_MAX, dtype=jnp.int32), side="right"),
        E - 1,
    ).astype(jnp.int32)
    counts_excl = jnp.cumsum(counts) - counts
    i = jnp.arange(S, dtype=jnp.int32)
    pos_sorted = TM * jnp.take(tcum_excl, sorted_e) + i - jnp.take(counts_excl, sorted_e)
    src_row = jnp.zeros((P,), jnp.int32).at[pos_sorted].set(order)
    pos = jnp.zeros((S,), jnp.int32).at[order].set(pos_sorted)
    return src_row, pos, tile_expert


def _sc_gather_in(x, src_row):
    mesh = plsc.VectorSubcoreMesh(core_axis_name="c", subcore_axis_name="s")

    @functools.partial(
        pl.kernel,
        mesh=mesh,
        out_type=jax.ShapeDtypeStruct((P, D), jnp.float32),
        scratch_types=[
            pltpu.VMEM((CHUNK,), jnp.int32),
            pltpu.VMEM((CHUNK,), jnp.int32),
            pltpu.VMEM((CHUNK,), jnp.int32),
            pltpu.VMEM((CHUNK,), jnp.int32),
            pltpu.VMEM((CHUNK,), jnp.int32),
            pltpu.VMEM((CHUNK, D), jnp.float32),
            pltpu.VMEM((CHUNK, D), jnp.float32),
            pltpu.SemaphoreType.DMA,
            pltpu.SemaphoreType.DMA,
            pltpu.SemaphoreType.DMA,
            pltpu.SemaphoreType.DMA,
        ],
    )
    def ka(x_hbm, src_hbm, xs_hbm, i0, i1, i2, i3, i4, r0, r1,
           gs0, gs1, ss0, ss1):
        wid = lax.axis_index("s") * NC + lax.axis_index("c")
        base = wid * ROWS_A
        idxs = [i0, i1, i2, i3, i4]
        for c in range(5):
            pltpu.sync_copy(src_hbm.at[pl.ds(base + c * CHUNK, CHUNK)], idxs[c])
        g0 = pltpu.async_copy(x_hbm.at[i0], r0, gs0)
        g1 = pltpu.async_copy(x_hbm.at[i1], r1, gs1)
        g0.wait()
        s0 = pltpu.async_copy(r0, xs_hbm.at[pl.ds(base, CHUNK)], ss0)
        g1.wait()
        s1 = pltpu.async_copy(r1, xs_hbm.at[pl.ds(base + CHUNK, CHUNK)], ss1)
        s0.wait()
        g2 = pltpu.async_copy(x_hbm.at[i2], r0, gs0)
        s1.wait()
        g3 = pltpu.async_copy(x_hbm.at[i3], r1, gs1)
        g2.wait()
        s2 = pltpu.async_copy(r0, xs_hbm.at[pl.ds(base + 2 * CHUNK, CHUNK)], ss0)
        g3.wait()
        s3 = pltpu.async_copy(r1, xs_hbm.at[pl.ds(base + 3 * CHUNK, CHUNK)], ss1)
        s2.wait()
        g4 = pltpu.async_copy(x_hbm.at[i4], r0, gs0)
        g4.wait()
        s4 = pltpu.async_copy(r0, xs_hbm.at[pl.ds(base + 4 * CHUNK, CHUNK)], ss0)
        s3.wait()
        s4.wait()

    return ka(x, src_row)


def _sc_gather_out(ys, pos):
    mesh = plsc.VectorSubcoreMesh(core_axis_name="c", subcore_axis_name="s")

    @functools.partial(
        pl.kernel,
        mesh=mesh,
        out_type=jax.ShapeDtypeStruct((S, D), jnp.float32),
        scratch_types=[
            pltpu.VMEM((ROWS_C,), jnp.int32),
            pltpu.VMEM((ROWS_C, D), jnp.float32),
            pltpu.SemaphoreType.DMA,
        ],
    )
    def kc(ys_hbm, pos_hbm, out_hbm, idx_v, rows_v, sem):
        wid = lax.axis_index("s") * NC + lax.axis_index("c")
        base = wid * ROWS_C
        pltpu.sync_copy(pos_hbm.at[pl.ds(base, ROWS_C)], idx_v)
        pltpu.async_copy(ys_hbm.at[idx_v], rows_v, sem).wait()
        pltpu.sync_copy(rows_v, out_hbm.at[pl.ds(base, ROWS_C)])

    return kc(ys, pos)


def _tc_body(te_ref, xs_ref, gu_ref, dn_ref, ys_ref):
    xt = xs_ref[...]
    gu = gu_ref[0]
    acts = lax.dot_general(
        xt, gu, (((1,), (1,)), ((), ())), preferred_element_type=jnp.float32
    )
    gate = acts[:, :F]
    up = acts[:, F:]
    act = gate * jax.nn.sigmoid(gate) * up
    dn = dn_ref[0]
    ys_ref[...] = lax.dot_general(
        act, dn, (((1,), (1,)), ((), ())), preferred_element_type=jnp.float32
    )


def _tc_grouped_matmul(xs, gate_up_weights, down_weights, tile_expert,
                       interpret=False):
    grid_spec = pltpu.PrefetchScalarGridSpec(
        num_scalar_prefetch=1,
        grid=(T_MAX,),
        in_specs=[
            pl.BlockSpec((TM, D), lambda j, te: (j, 0)),
            pl.BlockSpec((1, 2 * F, D), lambda j, te: (te[j], 0, 0)),
            pl.BlockSpec((1, D, F), lambda j, te: (te[j], 0, 0)),
        ],
        out_specs=pl.BlockSpec((TM, D), lambda j, te: (j, 0)),
    )
    return pl.pallas_call(
        _tc_body,
        grid_spec=grid_spec,
        out_shape=jax.ShapeDtypeStruct((P, D), jnp.float32),
        interpret=interpret,
    )(tile_expert, xs, gate_up_weights, down_weights)


def _scale_body(r_ref, w_ref, o_ref):
    o_ref[...] = r_ref[...] * w_ref[...]


def _tc_scale(rows, tw, interpret=False):
    return pl.pallas_call(
        _scale_body,
        out_shape=jax.ShapeDtypeStruct((S, D), jnp.float32),
        interpret=interpret,
    )(rows, tw.reshape(S, 1))


def kernel(x, weights, top_weights, top_experts, gate_up_weights, down_weights):
    del weights
    q_len = x.shape[1]
    xf = x.reshape(S, D)
    tw = top_weights[:, 0].astype(jnp.float32)

    src_row, pos, tile_expert = _routing_metadata(top_experts)
    xs = _sc_gather_in(xf, src_row)
    ys = _tc_grouped_matmul(xs, gate_up_weights, down_weights, tile_expert)
    rows = _sc_gather_out(ys, pos)
    out = _tc_scale(rows, tw)
    return out.reshape(-1, q_len, D)
